# baseline (speedup 1.0000x reference)
import jax
import jax.numpy as jnp
from jax.experimental import pallas as pl
from jax.experimental.pallas import tpu as pltpu

_N_FREQ_MASKS = 2
_FREQ_MASK_SIZE = 27.0
_N_TIME_MASKS = 2
_TIME_MASK_PCT = 0.05

_BBLK = 2


def _body(xlen_ref, frand_ref, trand_ref, x_ref, o_ref):
    gb = pl.program_id(0)
    _, T, F = x_ref.shape

    f_idx = jax.lax.broadcasted_iota(jnp.int32, (1, F), 1)
    fkeep = jnp.ones((1, F), jnp.float32)
    for i in range(_N_FREQ_MASKS):
        value = frand_ref[i, 0] * _FREQ_MASK_SIZE
        min_v = frand_ref[i, 1] * (jnp.float32(F) - value)
        start = jnp.floor(min_v)
        end = start + jnp.floor(value)
        band = (f_idx >= start.astype(jnp.int32)) & (f_idx < end.astype(jnp.int32))
        fkeep = jnp.where(band, jnp.float32(0.0), fkeep)

    t_idx = jax.lax.broadcasted_iota(jnp.int32, (T, 1), 0)
    for k in range(_BBLK):
        b = gb * _BBLK + k
        tkeep = jnp.ones((T, 1), jnp.float32)
        xlen_f = xlen_ref[b].astype(jnp.float32)
        param = jnp.floor(_TIME_MASK_PCT * xlen_f)
        for j in range(_N_TIME_MASKS):
            value = trand_ref[b, j, 0] * param
            min_v = trand_ref[b, j, 1] * (xlen_f - value)
            start = jnp.floor(min_v)
            end = start + jnp.floor(value)
            band = (t_idx >= start.astype(jnp.int32)) & (
                t_idx < end.astype(jnp.int32)
            )
            tkeep = jnp.where(band, jnp.float32(0.0), tkeep)
        o_ref[k] = x_ref[k] * (tkeep * fkeep)


def kernel(x, x_len, freq_rand, time_rand):
    B, T, F = x.shape
    return pl.pallas_call(
        _body,
        grid=(B // _BBLK,),
        in_specs=[
            pl.BlockSpec(memory_space=pltpu.SMEM),
            pl.BlockSpec(memory_space=pltpu.SMEM),
            pl.BlockSpec(memory_space=pltpu.SMEM),
            pl.BlockSpec((_BBLK, T, F), lambda b: (b, 0, 0)),
        ],
        out_specs=pl.BlockSpec((_BBLK, T, F), lambda b: (b, 0, 0)),
        out_shape=jax.ShapeDtypeStruct((B, T, F), x.dtype),
        compiler_params=pltpu.CompilerParams(
            dimension_semantics=("parallel",),
        ),
    )(x_len, freq_rand, time_rand, x)


# speedup vs baseline: 1.0623x; 1.0623x over previous
import jax
import jax.numpy as jnp
from jax.experimental import pallas as pl
from jax.experimental.pallas import tpu as pltpu

_N_FREQ_MASKS = 2
_FREQ_MASK_SIZE = 27.0
_N_TIME_MASKS = 2
_TIME_MASK_PCT = 0.05

_BBLK = 4
_W = 256


def _body(xlen_ref, frand_ref, trand_ref, x_ref, o_ref):
    gb = pl.program_id(0)
    _, T, F = x_ref.shape

    f_idx = jax.lax.broadcasted_iota(jnp.int32, (1, F), 1)
    fkeep = jnp.ones((1, F), jnp.float32)
    for i in range(_N_FREQ_MASKS):
        value = frand_ref[i, 0] * _FREQ_MASK_SIZE
        min_v = frand_ref[i, 1] * (jnp.float32(F) - value)
        start = jnp.floor(min_v)
        end = start + jnp.floor(value)
        band = (f_idx >= start.astype(jnp.int32)) & (f_idx < end.astype(jnp.int32))
        fkeep = jnp.where(band, jnp.float32(0.0), fkeep)

    w_idx = jax.lax.broadcasted_iota(jnp.int32, (_W, 1), 0)
    for k in range(_BBLK):
        b = gb * _BBLK + k
        o_ref[k] = x_ref[k] * fkeep
        xlen_f = xlen_ref[b].astype(jnp.float32)
        param = jnp.floor(_TIME_MASK_PCT * xlen_f)
        for j in range(_N_TIME_MASKS):
            value = trand_ref[b, j, 0] * param
            min_v = trand_ref[b, j, 1] * (xlen_f - value)
            start = jnp.floor(min_v)
            end = start + jnp.floor(value)
            istart = start.astype(jnp.int32)
            iend = end.astype(jnp.int32)
            base = jnp.minimum(istart, T - _W)
            t_global = w_idx + base
            band = (t_global >= istart) & (t_global < iend)
            win = o_ref[k, pl.ds(base, _W), :]
            o_ref[k, pl.ds(base, _W), :] = jnp.where(
                band, jnp.float32(0.0), win
            )


def kernel(x, x_len, freq_rand, time_rand):
    B, T, F = x.shape
    return pl.pallas_call(
        _body,
        grid=(B // _BBLK,),
        in_specs=[
            pl.BlockSpec(memory_space=pltpu.SMEM),
            pl.BlockSpec(memory_space=pltpu.SMEM),
            pl.BlockSpec(memory_space=pltpu.SMEM),
            pl.BlockSpec((_BBLK, T, F), lambda b: (b, 0, 0)),
        ],
        out_specs=pl.BlockSpec((_BBLK, T, F), lambda b: (b, 0, 0)),
        out_shape=jax.ShapeDtypeStruct((B, T, F), x.dtype),
        compiler_params=pltpu.CompilerParams(
            dimension_semantics=("parallel",),
        ),
    )(x_len, freq_rand, time_rand, x)
